# Initial kernel scaffold; baseline (speedup 1.0000x reference)
import jax
import jax.numpy as jnp
from jax.experimental import pallas as pl


def kernel(sentences, scalars):
    raise NotImplementedError("write your pallas kernel here")


# speedup vs baseline: 4.0856x; 4.0856x over previous
import functools

import jax
import jax.numpy as jnp
from jax import lax
from jax.experimental import pallas as pl
from jax.experimental.pallas import tpu as pltpu
from jax.experimental.pallas import tpu_sc as plsc

B = 256
L = 32
NC = 2
NS = 16
NW = NC * NS
BPW = B // NW
PAD = 64


def _cky_body(scalars_hbm, out_hbm, chart, stage0, stage1, res, sem):
    wid = lax.axis_index("s") * NC + lax.axis_index("c")
    b0 = wid * BPW

    iota = lax.iota(jnp.int32, 16)
    ones = jnp.ones((16,), jnp.float32)

    for b in range(BPW):
        chart[b, 0, pl.ds(0, 16)] = ones
        chart[b, 0, pl.ds(16, 16)] = ones

    stages = (stage0, stage1)
    pending = pltpu.async_copy(
        scalars_hbm.at[1, :, pl.ds(b0, BPW), :], stages[1], sem
    )

    for l in range(1, L):
        pending.wait()
        if l + 1 < L:
            pending = pltpu.async_copy(
                scalars_hbm.at[l + 1, :, pl.ds(b0, BPW), :],
                stages[(l + 1) & 1],
                sem,
            )
        stg = stages[l & 1]

        nblk = 2 if (32 - l) > 16 else 1
        for blk in range(nblk):
            p0 = 16 * blk
            p_idx = iota + p0
            b_idx = [jnp.full((16,), b, jnp.int32) for b in range(BPW)]

            def body(i, ms, p0=p0, stg=stg, lvl=l):
                i_vec = jnp.full((16,), i, jnp.int32)
                outs = []
                for b in range(BPW):
                    x = chart[b, i, pl.ds(p0, 16)]
                    y = chart[b, lvl - 1 - i, pl.ds(p0 + i + 1, 16)]
                    s = plsc.load_gather(stg, [p_idx, b_idx[b], i_vec])
                    outs.append(jnp.maximum(ms[b], x + y + s))
                return tuple(outs)

            init = tuple(
                jnp.full((16,), -jnp.inf, jnp.float32) for _ in range(BPW)
            )
            ms = lax.fori_loop(0, l, body, init)
            for b in range(BPW):
                chart[b, l, pl.ds(p0, 16)] = ms[b]

    lane_ok = iota < BPW
    vals = plsc.load_gather(
        chart,
        [iota, jnp.full((16,), L - 1, jnp.int32), jnp.zeros((16,), jnp.int32)],
        mask=lane_ok,
    )
    plsc.store_scatter(res, [iota], vals, mask=lane_ok)
    pltpu.sync_copy(res, out_hbm.at[pl.ds(b0, BPW)])


@jax.jit
def _cky_call(scalars):
    mesh = plsc.VectorSubcoreMesh(
        core_axis_name="c", subcore_axis_name="s", num_cores=NC, num_subcores=NS
    )
    return pl.kernel(
        _cky_body,
        out_type=jax.ShapeDtypeStruct((B,), jnp.float32),
        mesh=mesh,
        compiler_params=pltpu.CompilerParams(needs_layout_passes=False),
        scratch_types=[
            pltpu.VMEM((BPW, L, PAD), jnp.float32),
            pltpu.VMEM((L, BPW, L), jnp.float32),
            pltpu.VMEM((L, BPW, L), jnp.float32),
            pltpu.VMEM((BPW,), jnp.float32),
            pltpu.SemaphoreType.DMA,
        ],
    )(scalars)


def kernel(sentences, scalars):
    del sentences
    return _cky_call(scalars)


# speedup vs baseline: 8.7114x; 2.1322x over previous
import functools

import jax
import jax.numpy as jnp
from jax import lax
from jax.experimental import pallas as pl
from jax.experimental.pallas import tpu as pltpu
from jax.experimental.pallas import tpu_sc as plsc

B = 256
L = 32
NC = 2
NS = 16
LANES = 16
NG = B // LANES


def _cky_body(scalars_hbm, out_hbm, chart, stg0, stg1, res, sem):
    wid = lax.axis_index("s") * NC + lax.axis_index("c")

    @pl.when(wid < NG)
    def _():
        b0 = wid * LANES
        stages = (stg0, stg1)

        def issue(l):
            np_ = L - l
            return pltpu.async_copy(
                scalars_hbm.at[l, pl.ds(0, np_), pl.ds(0, l), pl.ds(b0, LANES)],
                stages[l & 1].at[pl.ds(0, np_), pl.ds(0, l), :],
                sem,
            )

        ones = jnp.ones((LANES,), jnp.float32)

        def init_p(p, _):
            chart[0, p, pl.ds(0, LANES)] = ones
            return 0

        lax.fori_loop(0, L, init_p, 0)

        pending = issue(1)

        for l in range(1, L):
            np_ = L - l
            pending.wait()
            if l + 1 < L:
                pending = issue(l + 1)
            stg = stages[l & 1]

            def cell(p, _, stg=stg, lvl=l):
                def split(i, m):
                    x = chart[i, p, pl.ds(0, LANES)]
                    y = chart[lvl - 1 - i, p + i + 1, pl.ds(0, LANES)]
                    s = stg[p, i, pl.ds(0, LANES)]
                    return jnp.maximum(m, x + y + s)

                m = lax.fori_loop(
                    0, lvl, split, jnp.full((LANES,), -jnp.inf, jnp.float32)
                )
                chart[lvl, p, pl.ds(0, LANES)] = m
                return 0

            lax.fori_loop(0, np_, cell, 0)

        res[pl.ds(0, LANES)] = chart[L - 1, 0, pl.ds(0, LANES)]
        pltpu.sync_copy(res, out_hbm.at[pl.ds(b0, LANES)])


@jax.jit
def _cky_call(scalars_t):
    mesh = plsc.VectorSubcoreMesh(
        core_axis_name="c", subcore_axis_name="s", num_cores=NC, num_subcores=NS
    )
    return pl.kernel(
        _cky_body,
        out_type=jax.ShapeDtypeStruct((B,), jnp.float32),
        mesh=mesh,
        compiler_params=pltpu.CompilerParams(
            needs_layout_passes=False, use_tc_tiling_on_sc=False
        ),
        scratch_types=[
            pltpu.VMEM((L, L, LANES), jnp.float32),
            pltpu.VMEM((L - 1, L - 1, LANES), jnp.float32),
            pltpu.VMEM((L - 1, L - 1, LANES), jnp.float32),
            pltpu.VMEM((LANES,), jnp.float32),
            pltpu.SemaphoreType.DMA,
        ],
    )(scalars_t)


def kernel(sentences, scalars):
    del sentences
    return _cky_call(jnp.transpose(scalars, (0, 1, 3, 2)))
